# baseline (speedup 1.0000x reference)
import functools
import jax
import jax.numpy as jnp
from jax import lax
from jax.experimental import pallas as pl
from jax.experimental.pallas import tpu as pltpu
from jax.experimental.pallas import tpu_sc as plsc

B = 4096
L = 200
D = 64
V = 200001
CBLK = 2048
VPAD = 200704
CH = 2
SLABL = 8
NCHUNK = L // CH
CPS = SLABL // CH
NSLAB = L // SLABL


def _norm_kernel(wt_ref, o_ref):
    w = wt_ref[...]
    s = jnp.sum(w * w, axis=0, keepdims=True)
    n = jnp.maximum(jnp.sqrt(s), 1e-12)
    wn = w / n
    eye = jnp.eye(D, dtype=jnp.float32)
    t = lax.dot_general(wn, eye, (((0,), (0,)), ((), ())),
                        preferred_element_type=jnp.float32)
    o_ref[:, pl.ds(0, D)] = t


def _normalize_table(wt):
    return pl.pallas_call(
        _norm_kernel,
        grid=(VPAD // CBLK,),
        in_specs=[pl.BlockSpec((D, CBLK), lambda i: (0, i))],
        out_specs=pl.BlockSpec((CBLK, 128), lambda i: (i, 0)),
        out_shape=jax.ShapeDtypeStruct((VPAD, 128), jnp.float32),
    )(wt)


def _sc_kernel(rid_hbm, e_hbm, tnorm_hbm, out_hbm,
               idx_v, w_v, e_v, acc_v, sems, slab_sem):
    info = plsc.get_sparse_core_info()
    nc = info.num_cores
    wid = lax.axis_index("s") * nc + lax.axis_index("c")

    lanes = jnp.arange(16, dtype=jnp.int32)
    rows_ug = [[jnp.full((16,), u * 128 + g * 16, jnp.int32) + lanes
                for g in range(8)] for u in range(CH)]

    def slab_start(si):
        return pltpu.async_copy(rid_hbm.at[si, wid], idx_v.at[si % 2],
                                slab_sem)

    def issue(k, slot):
        si = k // CPS
        for u in range(CH):
            pltpu.async_copy(
                tnorm_hbm.at[idx_v.at[si % 2, (k % CPS) * CH + u]],
                w_v.at[slot].at[pl.ds(u * 128, 128)], sems.at[slot])
        pltpu.async_copy(e_hbm.at[pl.ds(k * CH, CH), :, wid],
                         e_v.at[slot], sems.at[slot])

    def drain(slot):
        for u in range(CH):
            pltpu.make_async_copy(tnorm_hbm.at[pl.ds(0, 128)],
                                  w_v.at[slot].at[pl.ds(u * 128, 128)],
                                  sems.at[slot]).wait()
        pltpu.make_async_copy(e_hbm.at[pl.ds(0, CH), :, 0],
                              e_v.at[slot], sems.at[slot]).wait()

    def slab_wait():
        pltpu.make_async_copy(rid_hbm.at[0, 0], idx_v.at[0],
                              slab_sem).wait()

    z = jnp.zeros((16,), jnp.float32)

    def zero_body(dt, _):
        for ds_ in range(8):
            for g in range(8):
                acc_v[dt, ds_, pl.ds(g * 16, 16)] = z
        return 0

    lax.fori_loop(0, 8, zero_body, 0)

    def compute(slot, k):
        wb = w_v.at[slot]
        eb = e_v.at[slot]

        def p1_body(d, carry):
            cs = list(carry)
            dt = d // 8
            ds_ = d % 8
            dd = jnp.full((16,), d, jnp.int32)
            for u in range(CH):
                for g in range(8):
                    ev = eb[u, dt, ds_, pl.ds(g * 16, 16)]
                    wv = plsc.load_gather(wb, [rows_ug[u][g], dd])
                    cs[u * 8 + g] = cs[u * 8 + g] + ev * wv
            return tuple(cs)

        coefs = lax.fori_loop(0, D, p1_body, (z,) * (CH * 8))

        def p2_body(d, _):
            dt = d // 8
            ds_ = d % 8
            dd = jnp.full((16,), d, jnp.int32)
            for g in range(8):
                a = acc_v[dt, ds_, pl.ds(g * 16, 16)]
                for u in range(CH):
                    ev = eb[u, dt, ds_, pl.ds(g * 16, 16)]
                    wv = plsc.load_gather(wb, [rows_ug[u][g], dd])
                    a = a + (ev - coefs[u * 8 + g] * wv)
                acc_v[dt, ds_, pl.ds(g * 16, 16)] = a
            return 0

        lax.fori_loop(0, D, p2_body, 0)

    slab_start(0).wait()
    slab_start(1)
    issue(0, 0)

    def pair_body(p, _):
        for s2 in range(2):
            k = p * 2 + s2
            drain(s2)

            @pl.when(jnp.logical_and(k % CPS == CPS - 1, k + 1 < NCHUNK))
            def _():
                slab_wait()

            @pl.when(k + 1 < NCHUNK)
            def _():
                issue(k + 1, (s2 + 1) % 2)

            @pl.when(jnp.logical_and(k % CPS == CPS - 1,
                                     k // CPS + 2 < NSLAB))
            def _():
                slab_start(k // CPS + 2)

            compute(s2, k)
        return 0

    lax.fori_loop(0, NCHUNK // 2, pair_body, 0)

    inv = jnp.float32(1.0 / L)

    def scale_body(dt, _):
        for ds_ in range(8):
            for g in range(8):
                acc_v[dt, ds_, pl.ds(g * 16, 16)] = (
                    acc_v[dt, ds_, pl.ds(g * 16, 16)] * inv)
        return 0

    lax.fori_loop(0, 8, scale_body, 0)
    pltpu.sync_copy(acc_v, out_hbm.at[:, wid])


@jax.jit
def _run(rid4, e5, table_t):
    tnorm = _normalize_table(table_t)
    mesh = plsc.VectorSubcoreMesh(core_axis_name="c", subcore_axis_name="s")
    kfn = functools.partial(
        pl.kernel,
        mesh=mesh,
        compiler_params=pltpu.CompilerParams(use_tc_tiling_on_sc=False,
                                             needs_layout_passes=False),
        out_type=jax.ShapeDtypeStruct((8, 32, 8, 128), jnp.float32),
        scratch_types=[
            pltpu.VMEM((2, SLABL, 128), jnp.int32),
            pltpu.VMEM((2, CH * 128, 128), jnp.float32),
            pltpu.VMEM((2, CH, 8, 8, 128), jnp.float32),
            pltpu.VMEM((8, 8, 128), jnp.float32),
            pltpu.SemaphoreType.DMA((2,)),
            pltpu.SemaphoreType.DMA,
        ],
    )(_sc_kernel)
    return kfn(rid4, e5, tnorm)


def kernel(batch_nei_rid, batch_nei_e_emb, w_r_table):
    rid4 = (batch_nei_rid.T.reshape(25, 8, 32, 128)
            .transpose(0, 2, 1, 3))
    e5 = (jnp.transpose(batch_nei_e_emb, (1, 2, 0))
          .reshape(L, 8, 8, 32, 128)
          .transpose(0, 1, 3, 2, 4))
    table_t = w_r_table.T
    out4 = _run(rid4, e5, table_t)
    return out4.transpose(0, 2, 1, 3).reshape(D, B).T
